# baseline (speedup 1.0000x reference)
import functools

import jax
import jax.numpy as jnp
from jax import lax
from jax.experimental import pallas as pl
from jax.experimental.pallas import tpu as pltpu
from jax.experimental.pallas import tpu_sc as plsc

_VOCAB = 1000
_NTOK = 16384
_DPAD = 1024
_BLK = 128


@functools.cache
def _build():
    info = plsc.get_sparse_core_info()
    nc = info.num_cores
    nw = nc * info.num_subcores
    toks_per_w = _NTOK // nw
    n_chunks = toks_per_w // _BLK
    n_units = (_DPAD // _BLK) * n_chunks
    n_main = n_units - n_chunks
    tail_rows = _VOCAB - (_VOCAB // _BLK) * _BLK

    mesh = plsc.VectorSubcoreMesh(core_axis_name="c", subcore_axis_name="s")

    @functools.partial(
        pl.kernel,
        out_type=jax.ShapeDtypeStruct((_VOCAB, _NTOK), jnp.float32),
        mesh=mesh,
        compiler_params=pltpu.CompilerParams(use_tc_tiling_on_sc=True,
                                             needs_layout_passes=False),
        scratch_types=[
            pltpu.VMEM((toks_per_w,), jnp.int32),
            pltpu.VMEM((_BLK, _BLK), jnp.float32),
            pltpu.VMEM((_BLK, _BLK), jnp.float32),
            pltpu.VMEM((_BLK, _BLK), jnp.float32),
            pltpu.VMEM((_BLK, _BLK), jnp.float32),
            pltpu.SemaphoreType.DMA,
            pltpu.SemaphoreType.DMA,
            pltpu.SemaphoreType.DMA,
            pltpu.SemaphoreType.DMA,
        ],
    )
    def emb_kernel(x_hbm, table_hbm, out_t_hbm, idx_v, g0, g1, t0, t1,
                   gsem0, gsem1, ssem0, ssem1):
        wid = lax.axis_index("s") * nc + lax.axis_index("c")
        tok_base = wid * toks_per_w
        pltpu.sync_copy(x_hbm.at[pl.ds(tok_base, toks_per_w)], idx_v)

        def gather_refs(u, buf):
            k = u // n_chunks
            c = lax.rem(u, n_chunks)
            src = table_hbm.at[:, pl.ds(pl.multiple_of(k * _BLK, _BLK), _BLK)]
            return src.at[idx_v.at[pl.ds(c * _BLK, _BLK)]], buf

        def issue_gather(u, buf, sem):
            s, d = gather_refs(u, buf)
            pltpu.async_copy(s, d, sem)

        def wait_gather(u, buf, sem):
            s, d = gather_refs(u, buf)
            pltpu.make_async_copy(s, d, sem).wait()

        def store_refs(u, buf, rows):
            k = u // n_chunks
            c = lax.rem(u, n_chunks)
            dst = out_t_hbm.at[pl.ds(pl.multiple_of(k * _BLK, _BLK), rows),
                               pl.ds(tok_base + c * _BLK, _BLK)]
            return buf.at[pl.ds(0, rows), :], dst

        def issue_store(u, buf, sem, rows=_BLK):
            s, d = store_refs(u, buf, rows)
            pltpu.async_copy(s, d, sem)

        def wait_store(u, buf, sem, rows=_BLK):
            s, d = store_refs(u, buf, rows)
            pltpu.make_async_copy(s, d, sem).wait()

        iota16 = lax.iota(jnp.int32, 16)

        def transpose_block(gb, tb):
            def tg_body(tg, _):
                rowidx = iota16 + tg * 16

                def dg_body(dg, _):
                    dbase = dg * 16
                    for j in range(16):
                        dsplat = jnp.zeros((16,), jnp.int32) + (dbase + j)
                        vals = plsc.load_gather(gb, [rowidx, dsplat])
                        tb[dbase + j, pl.ds(tg * 16, 16)] = vals
                    return 0

                lax.fori_loop(0, _BLK // 16, dg_body, 0)
                return 0

            lax.fori_loop(0, _BLK // 16, tg_body, 0)

        issue_gather(0, g0, gsem0)

        def pair_body(p, _):
            u0 = 2 * p
            u1 = u0 + 1
            issue_gather(u1, g1, gsem1)
            wait_gather(u0, g0, gsem0)
            pl.when(p > 0)(lambda: wait_store(u0 - 2, t0, ssem0))
            transpose_block(g0, t0)
            issue_store(u0, t0, ssem0)
            issue_gather(u0 + 2, g0, gsem0)
            wait_gather(u1, g1, gsem1)
            pl.when(p > 0)(lambda: wait_store(u1 - 2, t1, ssem1))
            transpose_block(g1, t1)
            issue_store(u1, t1, ssem1)
            return 0

        lax.fori_loop(0, n_main // 2, pair_body, 0)

        def tail_loop(c, _):
            u = n_main + c
            wait_gather(u, g0, gsem0)
            pl.when(c == 0)(lambda: wait_store(n_main - 2, t0, ssem0))
            pl.when(c > 0)(lambda: wait_store(u - 1, t0, ssem0,
                                              rows=tail_rows))
            transpose_block(g0, t0)
            issue_store(u, t0, ssem0, rows=tail_rows)
            pl.when(c < n_chunks - 1)(lambda: issue_gather(u + 1, g0, gsem0))
            return 0

        lax.fori_loop(0, n_chunks, tail_loop, 0)

        wait_store(n_units - 1, t0, ssem0, rows=tail_rows)
        wait_store(n_main - 1, t1, ssem1)

    return emb_kernel


def kernel(x, table):
    table_p = jnp.pad(table, ((0, 0), (0, _DPAD - _VOCAB)))
    out_t = _build()(x, table_p)
    return out_t.T


# speedup vs baseline: 1.0112x; 1.0112x over previous
import functools

import jax
import jax.numpy as jnp
from jax import lax
from jax.experimental import pallas as pl
from jax.experimental.pallas import tpu as pltpu
from jax.experimental.pallas import tpu_sc as plsc

_VOCAB = 1000
_NTOK = 16384
_DPAD = 1024
_BLK = 128


@functools.cache
def _build():
    info = plsc.get_sparse_core_info()
    nc = info.num_cores
    nw = nc * info.num_subcores
    toks_per_w = _NTOK // nw
    n_chunks = toks_per_w // _BLK
    n_units = (_DPAD // _BLK) * n_chunks
    n_main = n_units - n_chunks
    tail_rows = _VOCAB - (_VOCAB // _BLK) * _BLK

    mesh = plsc.VectorSubcoreMesh(core_axis_name="c", subcore_axis_name="s")

    @functools.partial(
        pl.kernel,
        out_type=jax.ShapeDtypeStruct((_VOCAB, _NTOK), jnp.float32),
        mesh=mesh,
        compiler_params=pltpu.CompilerParams(use_tc_tiling_on_sc=True,
                                             needs_layout_passes=False),
        scratch_types=[
            pltpu.VMEM((toks_per_w,), jnp.int32),
            pltpu.VMEM((_BLK, _BLK), jnp.float32),
            pltpu.VMEM((_BLK, _BLK), jnp.float32),
            pltpu.VMEM((_BLK, _BLK), jnp.float32),
            pltpu.VMEM((_BLK, _BLK), jnp.float32),
            pltpu.SemaphoreType.DMA,
            pltpu.SemaphoreType.DMA,
            pltpu.SemaphoreType.DMA,
            pltpu.SemaphoreType.DMA,
        ],
    )
    def emb_kernel(x_hbm, table_hbm, out_t_hbm, idx_v, g0, g1, t0, t1,
                   gsem0, gsem1, ssem0, ssem1):
        wid = lax.axis_index("s") * nc + lax.axis_index("c")
        tok_base = wid * toks_per_w
        pltpu.sync_copy(x_hbm.at[pl.ds(tok_base, toks_per_w)], idx_v)

        def gather_refs(u, buf):
            k = u // n_chunks
            c = lax.rem(u, n_chunks)
            src = table_hbm.at[k]
            return src.at[idx_v.at[pl.ds(c * _BLK, _BLK)]], buf

        def issue_gather(u, buf, sem):
            s, d = gather_refs(u, buf)
            pltpu.async_copy(s, d, sem)

        def wait_gather(u, buf, sem):
            s, d = gather_refs(u, buf)
            pltpu.make_async_copy(s, d, sem).wait()

        def store_refs(u, buf, rows):
            k = u // n_chunks
            c = lax.rem(u, n_chunks)
            dst = out_t_hbm.at[pl.ds(pl.multiple_of(k * _BLK, _BLK), rows),
                               pl.ds(tok_base + c * _BLK, _BLK)]
            return buf.at[pl.ds(0, rows), :], dst

        def issue_store(u, buf, sem, rows=_BLK):
            s, d = store_refs(u, buf, rows)
            pltpu.async_copy(s, d, sem)

        def wait_store(u, buf, sem, rows=_BLK):
            s, d = store_refs(u, buf, rows)
            pltpu.make_async_copy(s, d, sem).wait()

        iota16 = lax.iota(jnp.int32, 16)

        def transpose_block(gb, tb):
            def tg_body(tg, _):
                rowidx = iota16 + tg * 16

                def dg_body(dg, _):
                    dbase = dg * 16
                    for j in range(16):
                        dsplat = jnp.zeros((16,), jnp.int32) + (dbase + j)
                        vals = plsc.load_gather(gb, [rowidx, dsplat])
                        tb[dbase + j, pl.ds(tg * 16, 16)] = vals
                    return 0

                lax.fori_loop(0, _BLK // 16, dg_body, 0)
                return 0

            lax.fori_loop(0, _BLK // 16, tg_body, 0)

        issue_gather(0, g0, gsem0)

        def pair_body(p, _):
            u0 = 2 * p
            u1 = u0 + 1
            issue_gather(u1, g1, gsem1)
            wait_gather(u0, g0, gsem0)
            pl.when(p > 0)(lambda: wait_store(u0 - 2, t0, ssem0))
            transpose_block(g0, t0)
            issue_store(u0, t0, ssem0)
            issue_gather(u0 + 2, g0, gsem0)
            wait_gather(u1, g1, gsem1)
            pl.when(p > 0)(lambda: wait_store(u1 - 2, t1, ssem1))
            transpose_block(g1, t1)
            issue_store(u1, t1, ssem1)
            return 0

        lax.fori_loop(0, n_main // 2, pair_body, 0)

        def tail_loop(c, _):
            u = n_main + c
            wait_gather(u, g0, gsem0)
            pl.when(c == 0)(lambda: wait_store(n_main - 2, t0, ssem0))
            pl.when(c > 0)(lambda: wait_store(u - 1, t0, ssem0,
                                              rows=tail_rows))
            transpose_block(g0, t0)
            issue_store(u, t0, ssem0, rows=tail_rows)
            pl.when(c < n_chunks - 1)(lambda: issue_gather(u + 1, g0, gsem0))
            return 0

        lax.fori_loop(0, n_chunks, tail_loop, 0)

        wait_store(n_units - 1, t0, ssem0, rows=tail_rows)
        wait_store(n_main - 1, t1, ssem1)

    return emb_kernel


def kernel(x, table):
    table_p = jnp.pad(table, ((0, 0), (0, _DPAD - _VOCAB)))
    table_3d = table_p.reshape(_VOCAB, _DPAD // _BLK, _BLK).transpose(1, 0, 2)
    out_t = _build()(x, table_3d)
    return out_t.T


# speedup vs baseline: 1.0135x; 1.0022x over previous
import functools

import jax
import jax.numpy as jnp
from jax import lax
from jax.experimental import pallas as pl
from jax.experimental.pallas import tpu as pltpu
from jax.experimental.pallas import tpu_sc as plsc

_VOCAB = 1000
_NTOK = 16384
_DPAD = 1024
_BLK = 128


@functools.cache
def _build():
    info = plsc.get_sparse_core_info()
    nc = info.num_cores
    nw = nc * info.num_subcores
    toks_per_w = _NTOK // nw
    n_chunks = toks_per_w // _BLK
    n_units = (_DPAD // _BLK) * n_chunks
    n_main = n_units - n_chunks
    tail_rows = _VOCAB - (_VOCAB // _BLK) * _BLK

    mesh = plsc.VectorSubcoreMesh(core_axis_name="c", subcore_axis_name="s")

    @functools.partial(
        pl.kernel,
        out_type=jax.ShapeDtypeStruct((_VOCAB, _NTOK), jnp.float32),
        mesh=mesh,
        compiler_params=pltpu.CompilerParams(use_tc_tiling_on_sc=True,
                                             needs_layout_passes=False),
        scratch_types=[
            pltpu.VMEM((toks_per_w,), jnp.int32),
            pltpu.VMEM((_BLK, _BLK), jnp.float32),
            pltpu.VMEM((_BLK, _BLK), jnp.float32),
            pltpu.VMEM((_BLK, _BLK), jnp.float32),
            pltpu.VMEM((_BLK, _BLK), jnp.float32),
            pltpu.SemaphoreType.DMA,
            pltpu.SemaphoreType.DMA,
            pltpu.SemaphoreType.DMA,
            pltpu.SemaphoreType.DMA,
        ],
    )
    def emb_kernel(x_hbm, table_hbm, out_t_hbm, idx_v, g0, g1, t0, t1,
                   gsem0, gsem1, ssem0, ssem1):
        wid = lax.axis_index("s") * nc + lax.axis_index("c")
        tok_base = wid * toks_per_w
        pltpu.sync_copy(x_hbm.at[pl.ds(tok_base, toks_per_w)], idx_v)

        def gather_refs(u, buf):
            k = u // n_chunks
            c = lax.rem(u, n_chunks)
            src = table_hbm.at[k]
            return src.at[idx_v.at[pl.ds(c * _BLK, _BLK)]], buf

        def issue_gather(u, buf, sem):
            s, d = gather_refs(u, buf)
            pltpu.async_copy(s, d, sem)

        def wait_gather(u, buf, sem):
            s, d = gather_refs(u, buf)
            pltpu.make_async_copy(s, d, sem).wait()

        def store_refs(u, buf, rows):
            k = u // n_chunks
            c = lax.rem(u, n_chunks)
            dst = out_t_hbm.at[pl.ds(pl.multiple_of(k * _BLK, _BLK), rows),
                               pl.ds(tok_base + c * _BLK, _BLK)]
            return buf.at[pl.ds(0, rows), :], dst

        def issue_store(u, buf, sem, rows=_BLK):
            s, d = store_refs(u, buf, rows)
            pltpu.async_copy(s, d, sem)

        def wait_store(u, buf, sem, rows=_BLK):
            s, d = store_refs(u, buf, rows)
            pltpu.make_async_copy(s, d, sem).wait()

        iota16 = lax.iota(jnp.int32, 16)

        def transpose_block(gb, tb):
            def tg_body(tg, _):
                rowidx = iota16 + tg * 16

                def dg_body(dg, dsp):
                    dbase = dg * 16
                    for j in range(16):
                        vals = plsc.load_gather(gb, [rowidx, dsp])
                        tb[dbase + j, pl.ds(tg * 16, 16)] = vals
                        dsp = dsp + 1
                    return dsp

                lax.fori_loop(0, _BLK // 16, dg_body,
                              jnp.zeros((16,), jnp.int32))
                return 0

            lax.fori_loop(0, _BLK // 16, tg_body, 0)

        issue_gather(0, g0, gsem0)

        def pair_body(p, _):
            u0 = 2 * p
            u1 = u0 + 1
            issue_gather(u1, g1, gsem1)
            wait_gather(u0, g0, gsem0)
            pl.when(p > 0)(lambda: wait_store(u0 - 2, t0, ssem0))
            transpose_block(g0, t0)
            issue_store(u0, t0, ssem0)
            issue_gather(u0 + 2, g0, gsem0)
            wait_gather(u1, g1, gsem1)
            pl.when(p > 0)(lambda: wait_store(u1 - 2, t1, ssem1))
            transpose_block(g1, t1)
            issue_store(u1, t1, ssem1)
            return 0

        lax.fori_loop(0, n_main // 2, pair_body, 0)

        def tail_loop(c, _):
            u = n_main + c
            wait_gather(u, g0, gsem0)
            pl.when(c == 0)(lambda: wait_store(n_main - 2, t0, ssem0))
            pl.when(c > 0)(lambda: wait_store(u - 1, t0, ssem0,
                                              rows=tail_rows))
            transpose_block(g0, t0)
            issue_store(u, t0, ssem0, rows=tail_rows)
            pl.when(c < n_chunks - 1)(lambda: issue_gather(u + 1, g0, gsem0))
            return 0

        lax.fori_loop(0, n_chunks, tail_loop, 0)

        wait_store(n_units - 1, t0, ssem0, rows=tail_rows)
        wait_store(n_main - 1, t1, ssem1)

    return emb_kernel


def kernel(x, table):
    table_p = jnp.pad(table, ((0, 0), (0, _DPAD - _VOCAB)))
    table_3d = table_p.reshape(_VOCAB, _DPAD // _BLK, _BLK).transpose(1, 0, 2)
    out_t = _build()(x, table_3d)
    return out_t.T


# speedup vs baseline: 1.7029x; 1.6803x over previous
import functools

import jax
import jax.numpy as jnp
from jax import lax
from jax.experimental import pallas as pl
from jax.experimental.pallas import tpu as pltpu
from jax.experimental.pallas import tpu_sc as plsc

_VOCAB = 1000
_NTOK = 16384
_DPAD = 1024
_BLK = 128


@functools.cache
def _build():
    info = plsc.get_sparse_core_info()
    nc = info.num_cores
    nw = nc * info.num_subcores
    toks_per_w = _NTOK // nw
    n_chunks = toks_per_w // _BLK
    n_units = (_DPAD // _BLK) * n_chunks
    n_main = n_units - n_chunks
    tail_rows = _VOCAB - (_VOCAB // _BLK) * _BLK

    mesh = plsc.VectorSubcoreMesh(core_axis_name="c", subcore_axis_name="s")

    @functools.partial(
        pl.kernel,
        out_type=jax.ShapeDtypeStruct((_VOCAB, _NTOK), jnp.float32),
        mesh=mesh,
        compiler_params=pltpu.CompilerParams(use_tc_tiling_on_sc=True,
                                             needs_layout_passes=False),
        scratch_types=[
            pltpu.VMEM((toks_per_w,), jnp.int32),
            pltpu.VMEM((_BLK, _BLK), jnp.float32),
            pltpu.VMEM((_BLK, _BLK), jnp.float32),
            pltpu.VMEM((_BLK, _BLK), jnp.float32),
            pltpu.VMEM((_BLK, _BLK), jnp.float32),
            pltpu.SemaphoreType.DMA,
            pltpu.SemaphoreType.DMA,
            pltpu.SemaphoreType.DMA,
            pltpu.SemaphoreType.DMA,
        ],
    )
    def emb_kernel(x_hbm, table_hbm, out_t_hbm, idx_v, g0, g1, t0, t1,
                   gsem0, gsem1, ssem0, ssem1):
        wid = lax.axis_index("s") * nc + lax.axis_index("c")
        tok_base = wid * toks_per_w
        pltpu.sync_copy(x_hbm.at[pl.ds(tok_base, toks_per_w)], idx_v)

        def gather_refs(u, buf):
            k = u // n_chunks
            c = lax.rem(u, n_chunks)
            src = table_hbm.at[k]
            return src.at[idx_v.at[pl.ds(c * _BLK, _BLK)]], buf

        def issue_gather(u, buf, sem):
            s, d = gather_refs(u, buf)
            pltpu.async_copy(s, d, sem)

        def wait_gather(u, buf, sem):
            s, d = gather_refs(u, buf)
            pltpu.make_async_copy(s, d, sem).wait()

        def store_refs(u, buf, rows):
            k = u // n_chunks
            c = lax.rem(u, n_chunks)
            dst = out_t_hbm.at[pl.ds(pl.multiple_of(k * _BLK, _BLK), rows),
                               pl.ds(tok_base + c * _BLK, _BLK)]
            return buf.at[pl.ds(0, rows), :], dst

        def issue_store(u, buf, sem, rows=_BLK):
            s, d = store_refs(u, buf, rows)
            pltpu.async_copy(s, d, sem)

        def wait_store(u, buf, sem, rows=_BLK):
            s, d = store_refs(u, buf, rows)
            pltpu.make_async_copy(s, d, sem).wait()

        iota16 = lax.iota(jnp.int32, 16)

        def transpose_block(gb, tb):
            def tg_body(tg, _):
                rowidx = iota16 + tg * 16
                tcol = tg * 16

                @plsc.parallel_loop(0, _BLK, 16)
                def dg_body(dbase):
                    base = jnp.zeros((16,), jnp.int32) + dbase
                    for j in range(16):
                        vals = plsc.load_gather(gb, [rowidx, base + j])
                        tb[dbase + j, pl.ds(tcol, 16)] = vals

                return 0

            lax.fori_loop(0, _BLK // 16, tg_body, 0)

        issue_gather(0, g0, gsem0)

        def pair_body(p, _):
            u0 = 2 * p
            u1 = u0 + 1
            issue_gather(u1, g1, gsem1)
            wait_gather(u0, g0, gsem0)
            pl.when(p > 0)(lambda: wait_store(u0 - 2, t0, ssem0))
            transpose_block(g0, t0)
            issue_store(u0, t0, ssem0)
            issue_gather(u0 + 2, g0, gsem0)
            wait_gather(u1, g1, gsem1)
            pl.when(p > 0)(lambda: wait_store(u1 - 2, t1, ssem1))
            transpose_block(g1, t1)
            issue_store(u1, t1, ssem1)
            return 0

        lax.fori_loop(0, n_main // 2, pair_body, 0)

        def tail_loop(c, _):
            u = n_main + c
            wait_gather(u, g0, gsem0)
            pl.when(c == 0)(lambda: wait_store(n_main - 2, t0, ssem0))
            pl.when(c > 0)(lambda: wait_store(u - 1, t0, ssem0,
                                              rows=tail_rows))
            transpose_block(g0, t0)
            issue_store(u, t0, ssem0, rows=tail_rows)
            pl.when(c < n_chunks - 1)(lambda: issue_gather(u + 1, g0, gsem0))
            return 0

        lax.fori_loop(0, n_chunks, tail_loop, 0)

        wait_store(n_units - 1, t0, ssem0, rows=tail_rows)
        wait_store(n_main - 1, t1, ssem1)

    return emb_kernel


def kernel(x, table):
    table_p = jnp.pad(table, ((0, 0), (0, _DPAD - _VOCAB)))
    table_3d = table_p.reshape(_VOCAB, _DPAD // _BLK, _BLK).transpose(1, 0, 2)
    out_t = _build()(x, table_3d)
    return out_t.T


# speedup vs baseline: 1.7749x; 1.0422x over previous
import functools

import jax
import jax.numpy as jnp
from jax import lax
from jax.experimental import pallas as pl
from jax.experimental.pallas import tpu as pltpu
from jax.experimental.pallas import tpu_sc as plsc

_VOCAB = 1000
_NTOK = 16384
_DPAD = 1024
_BLK = 128


@functools.cache
def _build():
    info = plsc.get_sparse_core_info()
    nc = info.num_cores
    nw = nc * info.num_subcores
    toks_per_w = _NTOK // nw
    n_chunks = toks_per_w // _BLK
    n_units = (_DPAD // _BLK) * n_chunks
    n_main = n_units - n_chunks
    tail_rows = _VOCAB - (_VOCAB // _BLK) * _BLK

    mesh = plsc.VectorSubcoreMesh(core_axis_name="c", subcore_axis_name="s")

    @functools.partial(
        pl.kernel,
        out_type=jax.ShapeDtypeStruct((_VOCAB, _NTOK), jnp.float32),
        mesh=mesh,
        compiler_params=pltpu.CompilerParams(use_tc_tiling_on_sc=True,
                                             needs_layout_passes=False),
        scratch_types=[
            pltpu.VMEM((toks_per_w,), jnp.int32),
            pltpu.VMEM((_BLK, _BLK), jnp.float32),
            pltpu.VMEM((_BLK, _BLK), jnp.float32),
            pltpu.VMEM((_BLK, _BLK), jnp.float32),
            pltpu.VMEM((_BLK, _BLK), jnp.float32),
            pltpu.SemaphoreType.DMA,
            pltpu.SemaphoreType.DMA,
            pltpu.SemaphoreType.DMA,
            pltpu.SemaphoreType.DMA,
        ],
    )
    def emb_kernel(x_hbm, table_hbm, out_t_hbm, idx_v, g0, g1, t0, t1,
                   gsem0, gsem1, ssem0, ssem1):
        wid = lax.axis_index("s") * nc + lax.axis_index("c")
        tok_base = wid * toks_per_w
        pltpu.sync_copy(x_hbm.at[pl.ds(tok_base, toks_per_w)], idx_v)

        def gather_refs(u, buf):
            k = u // n_chunks
            c = lax.rem(u, n_chunks)
            src = table_hbm.at[k]
            return src.at[idx_v.at[pl.ds(c * _BLK, _BLK)]], buf

        def issue_gather(u, buf, sem):
            s, d = gather_refs(u, buf)
            pltpu.async_copy(s, d, sem)

        def wait_gather(u, buf, sem):
            s, d = gather_refs(u, buf)
            pltpu.make_async_copy(s, d, sem).wait()

        def store_refs(u, buf, rows):
            k = u // n_chunks
            c = lax.rem(u, n_chunks)
            dst = out_t_hbm.at[pl.ds(pl.multiple_of(k * _BLK, _BLK), rows),
                               pl.ds(tok_base + c * _BLK, _BLK)]
            return buf.at[pl.ds(0, rows), :], dst

        def issue_store(u, buf, sem, rows=_BLK):
            s, d = store_refs(u, buf, rows)
            pltpu.async_copy(s, d, sem)

        def wait_store(u, buf, sem, rows=_BLK):
            s, d = store_refs(u, buf, rows)
            pltpu.make_async_copy(s, d, sem).wait()

        iota16 = lax.iota(jnp.int32, 16)

        def transpose_block(gb, tb):
            @plsc.parallel_loop(0, (_BLK // 16) * (_BLK // 16), 1, unroll=2)
            def sub_body(i):
                tg = lax.shift_right_logical(i, 3)
                dg = lax.bitwise_and(i, 7)
                rowidx = iota16 + tg * 16
                tcol = tg * 16
                dbase = dg * 16
                base = jnp.zeros((16,), jnp.int32) + dbase
                for j in range(16):
                    vals = plsc.load_gather(gb, [rowidx, base + j])
                    tb[dbase + j, pl.ds(tcol, 16)] = vals

        issue_gather(0, g0, gsem0)

        def pair_body(p, _):
            u0 = 2 * p
            u1 = u0 + 1
            issue_gather(u1, g1, gsem1)
            wait_gather(u0, g0, gsem0)
            pl.when(p > 0)(lambda: wait_store(u0 - 2, t0, ssem0))
            transpose_block(g0, t0)
            issue_store(u0, t0, ssem0)
            issue_gather(u0 + 2, g0, gsem0)
            wait_gather(u1, g1, gsem1)
            pl.when(p > 0)(lambda: wait_store(u1 - 2, t1, ssem1))
            transpose_block(g1, t1)
            issue_store(u1, t1, ssem1)
            return 0

        lax.fori_loop(0, n_main // 2, pair_body, 0)

        def tail_loop(c, _):
            u = n_main + c
            wait_gather(u, g0, gsem0)
            pl.when(c == 0)(lambda: wait_store(n_main - 2, t0, ssem0))
            pl.when(c > 0)(lambda: wait_store(u - 1, t0, ssem0,
                                              rows=tail_rows))
            transpose_block(g0, t0)
            issue_store(u, t0, ssem0, rows=tail_rows)
            pl.when(c < n_chunks - 1)(lambda: issue_gather(u + 1, g0, gsem0))
            return 0

        lax.fori_loop(0, n_chunks, tail_loop, 0)

        wait_store(n_units - 1, t0, ssem0, rows=tail_rows)
        wait_store(n_main - 1, t1, ssem1)

    return emb_kernel


def kernel(x, table):
    table_p = jnp.pad(table, ((0, 0), (0, _DPAD - _VOCAB)))
    table_3d = table_p.reshape(_VOCAB, _DPAD // _BLK, _BLK).transpose(1, 0, 2)
    out_t = _build()(x, table_3d)
    return out_t.T


# speedup vs baseline: 4.3396x; 2.4450x over previous
import functools

import jax
import jax.numpy as jnp
from jax import lax
from jax.experimental import pallas as pl
from jax.experimental.pallas import tpu as pltpu
from jax.experimental.pallas import tpu_sc as plsc

_VOCAB = 1000
_NTOK = 16384
_DPAD = 1024
_BLK = 128


@functools.cache
def _build():
    info = plsc.get_sparse_core_info()
    nc = info.num_cores
    nw = nc * info.num_subcores
    toks_per_w = _NTOK // nw
    n_chunks = toks_per_w // _BLK
    n_units = (_DPAD // _BLK) * n_chunks
    n_main = n_units - n_chunks
    tail_rows = _VOCAB - (_VOCAB // _BLK) * _BLK

    mesh = plsc.VectorSubcoreMesh(core_axis_name="c", subcore_axis_name="s")

    @functools.partial(
        pl.kernel,
        out_type=jax.ShapeDtypeStruct((_VOCAB, _NTOK), jnp.float32),
        mesh=mesh,
        compiler_params=pltpu.CompilerParams(use_tc_tiling_on_sc=True,
                                             needs_layout_passes=False),
        scratch_types=[
            pltpu.VMEM((toks_per_w,), jnp.int32),
            pltpu.VMEM((_BLK, _BLK), jnp.float32),
            pltpu.VMEM((_BLK, _BLK), jnp.float32),
            pltpu.VMEM((_BLK, _BLK), jnp.float32),
            pltpu.VMEM((_BLK, _BLK), jnp.float32),
            pltpu.SemaphoreType.DMA,
            pltpu.SemaphoreType.DMA,
            pltpu.SemaphoreType.DMA,
            pltpu.SemaphoreType.DMA,
        ],
    )
    def emb_kernel(x_hbm, table_hbm, out_t_hbm, idx_v, g0, g1, t0, t1,
                   gsem0, gsem1, ssem0, ssem1):
        wid = lax.axis_index("s") * nc + lax.axis_index("c")
        tok_base = wid * toks_per_w
        pltpu.sync_copy(x_hbm.at[pl.ds(tok_base, toks_per_w)], idx_v)

        def gather_refs(u, buf):
            k = u // n_chunks
            c = lax.rem(u, n_chunks)
            src = table_hbm.at[k]
            return src.at[idx_v.at[pl.ds(c * _BLK, _BLK)]], buf

        def issue_gather(u, buf, sem):
            s, d = gather_refs(u, buf)
            pltpu.async_copy(s, d, sem)

        def wait_gather(u, buf, sem):
            s, d = gather_refs(u, buf)
            pltpu.make_async_copy(s, d, sem).wait()

        def store_refs(u, buf, rows):
            k = u // n_chunks
            c = lax.rem(u, n_chunks)
            dst = out_t_hbm.at[pl.ds(pl.multiple_of(k * _BLK, _BLK), rows),
                               pl.ds(tok_base + c * _BLK, _BLK)]
            return buf.at[pl.ds(0, rows), :], dst

        def issue_store(u, buf, sem, rows=_BLK):
            s, d = store_refs(u, buf, rows)
            pltpu.async_copy(s, d, sem)

        def wait_store(u, buf, sem, rows=_BLK):
            s, d = store_refs(u, buf, rows)
            pltpu.make_async_copy(s, d, sem).wait()

        iota16 = lax.iota(jnp.int32, 16)

        def transpose_block(gb, tb):
            @plsc.parallel_loop(0, (_BLK // 16) * (_BLK // 16), 1, unroll=2)
            def sub_body(i):
                tg = lax.shift_right_logical(i, 3)
                dg = lax.bitwise_and(i, 7)
                rowidx = iota16 + tg * 16
                dbase = dg * 16
                base = jnp.zeros((16,), jnp.int32) + dbase
                rot = iota16
                for j in range(16):
                    col = base + rot
                    vals = plsc.load_gather(gb, [rowidx, col])
                    plsc.store_scatter(tb, [col, rowidx], vals)
                    rot = lax.bitwise_and(rot + 1, 15)

        issue_gather(0, g0, gsem0)

        def pair_body(p, _):
            u0 = 2 * p
            u1 = u0 + 1
            issue_gather(u1, g1, gsem1)
            wait_gather(u0, g0, gsem0)
            pl.when(p > 0)(lambda: wait_store(u0 - 2, t0, ssem0))
            transpose_block(g0, t0)
            issue_store(u0, t0, ssem0)
            issue_gather(u0 + 2, g0, gsem0)
            wait_gather(u1, g1, gsem1)
            pl.when(p > 0)(lambda: wait_store(u1 - 2, t1, ssem1))
            transpose_block(g1, t1)
            issue_store(u1, t1, ssem1)
            return 0

        lax.fori_loop(0, n_main // 2, pair_body, 0)

        def tail_loop(c, _):
            u = n_main + c
            wait_gather(u, g0, gsem0)
            pl.when(c == 0)(lambda: wait_store(n_main - 2, t0, ssem0))
            pl.when(c > 0)(lambda: wait_store(u - 1, t0, ssem0,
                                              rows=tail_rows))
            transpose_block(g0, t0)
            issue_store(u, t0, ssem0, rows=tail_rows)
            pl.when(c < n_chunks - 1)(lambda: issue_gather(u + 1, g0, gsem0))
            return 0

        lax.fori_loop(0, n_chunks, tail_loop, 0)

        wait_store(n_units - 1, t0, ssem0, rows=tail_rows)
        wait_store(n_main - 1, t1, ssem1)

    return emb_kernel


def kernel(x, table):
    table_p = jnp.pad(table, ((0, 0), (0, _DPAD - _VOCAB)))
    table_3d = table_p.reshape(_VOCAB, _DPAD // _BLK, _BLK).transpose(1, 0, 2)
    out_t = _build()(x, table_3d)
    return out_t.T


# speedup vs baseline: 4.5095x; 1.0391x over previous
import functools

import jax
import jax.numpy as jnp
from jax import lax
from jax.experimental import pallas as pl
from jax.experimental.pallas import tpu as pltpu
from jax.experimental.pallas import tpu_sc as plsc

_VOCAB = 1000
_NTOK = 16384
_DPAD = 1024
_BLK = 128


@functools.cache
def _build():
    info = plsc.get_sparse_core_info()
    nc = info.num_cores
    nw = nc * info.num_subcores
    toks_per_w = _NTOK // nw
    n_chunks = toks_per_w // _BLK
    n_units = (_DPAD // _BLK) * n_chunks
    n_main = n_units - n_chunks
    tail_rows = _VOCAB - (_VOCAB // _BLK) * _BLK

    mesh = plsc.VectorSubcoreMesh(core_axis_name="c", subcore_axis_name="s")

    @functools.partial(
        pl.kernel,
        out_type=jax.ShapeDtypeStruct((_VOCAB, _NTOK), jnp.float32),
        mesh=mesh,
        compiler_params=pltpu.CompilerParams(use_tc_tiling_on_sc=True,
                                             needs_layout_passes=False),
        scratch_types=[
            pltpu.VMEM((toks_per_w,), jnp.int32),
            pltpu.VMEM((_BLK, _BLK), jnp.float32),
            pltpu.VMEM((_BLK, _BLK), jnp.float32),
            pltpu.VMEM((_BLK, _BLK), jnp.float32),
            pltpu.VMEM((_BLK, _BLK), jnp.float32),
            pltpu.SemaphoreType.DMA,
            pltpu.SemaphoreType.DMA,
            pltpu.SemaphoreType.DMA,
            pltpu.SemaphoreType.DMA,
        ],
    )
    def emb_kernel(x_hbm, table_hbm, out_t_hbm, idx_v, g0, g1, t0, t1,
                   gsem0, gsem1, ssem0, ssem1):
        wid = lax.axis_index("s") * nc + lax.axis_index("c")
        tok_base = wid * toks_per_w
        pltpu.sync_copy(x_hbm.at[pl.ds(tok_base, toks_per_w)], idx_v)

        def gather_refs(u, buf):
            k = u // n_chunks
            c = lax.rem(u, n_chunks)
            src = table_hbm.at[k]
            return src.at[idx_v.at[pl.ds(c * _BLK, _BLK)]], buf

        def issue_gather(u, buf, sem):
            s, d = gather_refs(u, buf)
            pltpu.async_copy(s, d, sem)

        def wait_gather(u, buf, sem):
            s, d = gather_refs(u, buf)
            pltpu.make_async_copy(s, d, sem).wait()

        def store_refs(u, buf, rows):
            k = u // n_chunks
            c = lax.rem(u, n_chunks)
            dst = out_t_hbm.at[pl.ds(pl.multiple_of(k * _BLK, _BLK), rows),
                               pl.ds(tok_base + c * _BLK, _BLK)]
            return buf.at[pl.ds(0, rows), :], dst

        def issue_store(u, buf, sem, rows=_BLK):
            s, d = store_refs(u, buf, rows)
            pltpu.async_copy(s, d, sem)

        def wait_store(u, buf, sem, rows=_BLK):
            s, d = store_refs(u, buf, rows)
            pltpu.make_async_copy(s, d, sem).wait()

        iota16 = lax.iota(jnp.int32, 16)

        zero16 = jnp.zeros((16,), jnp.int32)
        iota128 = iota16 * _BLK
        rots = tuple(lax.bitwise_and(iota16 + j, 15) for j in range(16))
        rshifts = tuple(r * _BLK for r in rots)

        def transpose_block(gb, tb):
            @plsc.parallel_loop(0, (_BLK // 16) * (_BLK // 16), 1, unroll=2)
            def sub_body(i):
                tg = lax.shift_right_logical(i, 3)
                dg = lax.bitwise_and(i, 7)
                tcol = tg * 16
                dbase = dg * 16
                lbase = (zero16 + (tcol * _BLK + dbase)) + iota16
                sbase = (zero16 + (dbase * _BLK + tcol)) + iota128
                for j in range(16):
                    vals = plsc.load_gather(gb, [zero16, lbase + rshifts[j]])
                    plsc.store_scatter(tb, [zero16, sbase + rots[j]], vals)

        issue_gather(0, g0, gsem0)

        def pair_body(p, _):
            u0 = 2 * p
            u1 = u0 + 1
            issue_gather(u1, g1, gsem1)
            wait_gather(u0, g0, gsem0)
            pl.when(p > 0)(lambda: wait_store(u0 - 2, t0, ssem0))
            transpose_block(g0, t0)
            issue_store(u0, t0, ssem0)
            issue_gather(u0 + 2, g0, gsem0)
            wait_gather(u1, g1, gsem1)
            pl.when(p > 0)(lambda: wait_store(u1 - 2, t1, ssem1))
            transpose_block(g1, t1)
            issue_store(u1, t1, ssem1)
            return 0

        lax.fori_loop(0, n_main // 2, pair_body, 0)

        def tail_loop(c, _):
            u = n_main + c
            wait_gather(u, g0, gsem0)
            pl.when(c == 0)(lambda: wait_store(n_main - 2, t0, ssem0))
            pl.when(c > 0)(lambda: wait_store(u - 1, t0, ssem0,
                                              rows=tail_rows))
            transpose_block(g0, t0)
            issue_store(u, t0, ssem0, rows=tail_rows)
            pl.when(c < n_chunks - 1)(lambda: issue_gather(u + 1, g0, gsem0))
            return 0

        lax.fori_loop(0, n_chunks, tail_loop, 0)

        wait_store(n_units - 1, t0, ssem0, rows=tail_rows)
        wait_store(n_main - 1, t1, ssem1)

    return emb_kernel


def kernel(x, table):
    table_p = jnp.pad(table, ((0, 0), (0, _DPAD - _VOCAB)))
    table_3d = table_p.reshape(_VOCAB, _DPAD // _BLK, _BLK).transpose(1, 0, 2)
    out_t = _build()(x, table_3d)
    return out_t.T


# speedup vs baseline: 4.5144x; 1.0011x over previous
import functools

import jax
import jax.numpy as jnp
from jax import lax
from jax.experimental import pallas as pl
from jax.experimental.pallas import tpu as pltpu
from jax.experimental.pallas import tpu_sc as plsc

_VOCAB = 1000
_NTOK = 16384
_DPAD = 1024
_BLK = 128


@functools.cache
def _build():
    info = plsc.get_sparse_core_info()
    nc = info.num_cores
    nw = nc * info.num_subcores
    toks_per_w = _NTOK // nw
    n_chunks = toks_per_w // _BLK
    n_units = (_DPAD // _BLK) * n_chunks
    n_main = n_units - n_chunks
    tail_rows = _VOCAB - (_VOCAB // _BLK) * _BLK

    mesh = plsc.VectorSubcoreMesh(core_axis_name="c", subcore_axis_name="s")

    @functools.partial(
        pl.kernel,
        out_type=jax.ShapeDtypeStruct((_VOCAB, _NTOK), jnp.float32),
        mesh=mesh,
        compiler_params=pltpu.CompilerParams(use_tc_tiling_on_sc=True,
                                             needs_layout_passes=False),
        scratch_types=[
            pltpu.VMEM((toks_per_w,), jnp.int32),
            pltpu.VMEM((_BLK, _BLK), jnp.float32),
            pltpu.VMEM((_BLK, _BLK), jnp.float32),
            pltpu.VMEM((_BLK, _BLK), jnp.float32),
            pltpu.VMEM((_BLK, _BLK), jnp.float32),
            pltpu.SemaphoreType.DMA,
            pltpu.SemaphoreType.DMA,
            pltpu.SemaphoreType.DMA,
            pltpu.SemaphoreType.DMA,
        ],
    )
    def emb_kernel(x_hbm, table_hbm, out_t_hbm, idx_v, g0, g1, t0, t1,
                   gsem0, gsem1, ssem0, ssem1):
        wid = lax.axis_index("s") * nc + lax.axis_index("c")
        tok_base = wid * toks_per_w
        pltpu.sync_copy(x_hbm.at[pl.ds(tok_base, toks_per_w)], idx_v)

        def gather_refs(u, buf):
            k = u // n_chunks
            c = lax.rem(u, n_chunks)
            src = table_hbm.at[k]
            return src.at[idx_v.at[pl.ds(c * _BLK, _BLK)]], buf

        def issue_gather(u, buf, sem):
            s, d = gather_refs(u, buf)
            pltpu.async_copy(s, d, sem)

        def wait_gather(u, buf, sem):
            s, d = gather_refs(u, buf)
            pltpu.make_async_copy(s, d, sem).wait()

        def store_refs(u, buf, rows):
            k = u // n_chunks
            c = lax.rem(u, n_chunks)
            dst = out_t_hbm.at[pl.ds(pl.multiple_of(k * _BLK, _BLK), rows),
                               pl.ds(tok_base + c * _BLK, _BLK)]
            return buf.at[pl.ds(0, rows), :], dst

        def issue_store(u, buf, sem, rows=_BLK):
            s, d = store_refs(u, buf, rows)
            pltpu.async_copy(s, d, sem)

        def wait_store(u, buf, sem, rows=_BLK):
            s, d = store_refs(u, buf, rows)
            pltpu.make_async_copy(s, d, sem).wait()

        iota16 = lax.iota(jnp.int32, 16)

        zero16 = jnp.zeros((16,), jnp.int32)
        iota128 = iota16 * _BLK
        rots = tuple(lax.bitwise_and(iota16 + j, 15) for j in range(16))
        rshifts = tuple(r * _BLK for r in rots)

        def transpose_block(gb, tb):
            @plsc.parallel_loop(0, (_BLK // 16) * (_BLK // 16), 1, unroll=4)
            def sub_body(i):
                tg = lax.shift_right_logical(i, 3)
                dg = lax.bitwise_and(i, 7)
                tcol = tg * 16
                dbase = dg * 16
                lbase = (zero16 + (tcol * _BLK + dbase)) + iota16
                sbase = (zero16 + (dbase * _BLK + tcol)) + iota128
                for j in range(16):
                    vals = plsc.load_gather(gb, [zero16, lbase + rshifts[j]])
                    plsc.store_scatter(tb, [zero16, sbase + rots[j]], vals)

        issue_gather(0, g0, gsem0)

        def pair_body(p, _):
            u0 = 2 * p
            u1 = u0 + 1
            issue_gather(u1, g1, gsem1)
            wait_gather(u0, g0, gsem0)
            pl.when(p > 0)(lambda: wait_store(u0 - 2, t0, ssem0))
            transpose_block(g0, t0)
            issue_store(u0, t0, ssem0)
            issue_gather(u0 + 2, g0, gsem0)
            wait_gather(u1, g1, gsem1)
            pl.when(p > 0)(lambda: wait_store(u1 - 2, t1, ssem1))
            transpose_block(g1, t1)
            issue_store(u1, t1, ssem1)
            return 0

        lax.fori_loop(0, n_main // 2, pair_body, 0)

        def tail_loop(c, _):
            u = n_main + c
            wait_gather(u, g0, gsem0)
            pl.when(c == 0)(lambda: wait_store(n_main - 2, t0, ssem0))
            pl.when(c > 0)(lambda: wait_store(u - 1, t0, ssem0,
                                              rows=tail_rows))
            transpose_block(g0, t0)
            issue_store(u, t0, ssem0, rows=tail_rows)
            pl.when(c < n_chunks - 1)(lambda: issue_gather(u + 1, g0, gsem0))
            return 0

        lax.fori_loop(0, n_chunks, tail_loop, 0)

        wait_store(n_units - 1, t0, ssem0, rows=tail_rows)
        wait_store(n_main - 1, t1, ssem1)

    return emb_kernel


def kernel(x, table):
    table_p = jnp.pad(table, ((0, 0), (0, _DPAD - _VOCAB)))
    table_3d = table_p.reshape(_VOCAB, _DPAD // _BLK, _BLK).transpose(1, 0, 2)
    out_t = _build()(x, table_3d)
    return out_t.T
